# Initial kernel scaffold; baseline (speedup 1.0000x reference)
import functools

import jax
import jax.numpy as jnp
from jax import lax
from jax.experimental import pallas as pl
from jax.experimental.pallas import tpu as pltpu
from jax.experimental.pallas import tpu_sc as plsc

N = 10000
E = 320000
D = 128

NC = 2
NS = 16
L = 16

EPW = E // NS
K = 80
NCHUNK = EPW // K
RPT = N // NS
CW = 16


def _sc_body(x_hbm, ei_hbm, acc_hbm, cnt_hbm,
             acc_sh, cnt_sh, gidx, sidx, rows, ones, zrow, zcnt, sem):
    c = lax.axis_index("c")
    s = lax.axis_index("s")

    z16 = jnp.zeros((L,), jnp.float32)
    for j in range(D // L):
        zrow[0, pl.ds(j * L, L)] = z16
    for w in (1, 2, 4, 8, 16, 32, 64):
        pltpu.sync_copy(zrow.at[pl.ds(0, w)], zrow.at[pl.ds(w, w)])
    zcnt[0, pl.ds(0, L)] = z16
    for w in (1, 2, 4, 8, 16, 32, 64, 128, 256):
        pltpu.sync_copy(zcnt.at[pl.ds(0, w)], zcnt.at[pl.ds(w, w)])
    pltpu.sync_copy(zcnt.at[pl.ds(0, 128)], zcnt.at[pl.ds(512, 128)])
    o16 = jnp.ones((L,), jnp.float32)
    for r in range(K):
        ones[r, pl.ds(0, L)] = o16

    r0 = s * RPT
    for m in range(4):
        pltpu.sync_copy(zrow, acc_sh.at[pl.ds(r0 + m * 128, 128)])
    pltpu.sync_copy(zrow.at[pl.ds(0, RPT - 512)],
                    acc_sh.at[pl.ds(r0 + 512, RPT - 512)])
    pltpu.sync_copy(zcnt.at[pl.ds(0, RPT)], cnt_sh.at[pl.ds(r0, RPT)])
    plsc.subcore_barrier()

    def step(j, carry):
        base = s * EPW + j * K
        pltpu.sync_copy(ei_hbm.at[c, pl.ds(base, K)], gidx)
        pltpu.sync_copy(ei_hbm.at[1 - c, pl.ds(base, K)], sidx)
        pltpu.async_copy(x_hbm.at[gidx], rows, sem).wait()
        pltpu.sync_copy(rows, acc_sh.at[sidx], add=True)
        pltpu.sync_copy(ones, cnt_sh.at[sidx], add=True)
        return carry

    lax.fori_loop(0, NCHUNK, step, 0)
    plsc.subcore_barrier()

    pltpu.sync_copy(acc_sh.at[pl.ds(r0, RPT)], acc_hbm.at[c, pl.ds(r0, RPT)])
    pltpu.sync_copy(cnt_sh.at[pl.ds(r0, RPT)], cnt_hbm.at[c, pl.ds(r0, RPT)])


@functools.partial(
    pl.kernel,
    out_type=(jax.ShapeDtypeStruct((NC, N, D), jnp.float32),
              jax.ShapeDtypeStruct((NC, N, CW), jnp.float32)),
    mesh=plsc.VectorSubcoreMesh(core_axis_name="c", subcore_axis_name="s"),
    scratch_types=[
        pltpu.VMEM_SHARED((N, D), jnp.float32),
        pltpu.VMEM_SHARED((N, CW), jnp.float32),
        pltpu.VMEM((K,), jnp.int32),
        pltpu.VMEM((K,), jnp.int32),
        pltpu.VMEM((K, D), jnp.float32),
        pltpu.VMEM((K, CW), jnp.float32),
        pltpu.VMEM((128, D), jnp.float32),
        pltpu.VMEM((640, CW), jnp.float32),
        pltpu.SemaphoreType.DMA,
    ],
)
def _sc_aggregate(x_hbm, ei_hbm, acc_hbm, cnt_hbm, *scratch):
    _sc_body(x_hbm, ei_hbm, acc_hbm, cnt_hbm, *scratch)


R = 400


def _tc_body(acc_i, cnt_i, acc_o, cnt_o, x_ref,
             wli, wlo, wri, wro, bli, blo, out_ref):
    mi = acc_i[...] / jnp.maximum(cnt_i[:, 0:1], 1.0)
    mo = acc_o[...] / jnp.maximum(cnt_o[:, 0:1], 1.0)
    o = jnp.dot(mi, wli[...], preferred_element_type=jnp.float32)
    o = o + jnp.dot(mo, wlo[...], preferred_element_type=jnp.float32)
    o = o + jnp.dot(x_ref[...], wri[...] + wro[...],
                    preferred_element_type=jnp.float32)
    out_ref[...] = o + bli[...] + blo[...]


def _tc_combine(acc_in, cnt_in, acc_out, cnt_out, x,
                wli_t, wlo_t, wri_t, wro_t, bli, blo):
    blk = lambda w: pl.BlockSpec((R, w), lambda i: (i, 0))
    full = pl.BlockSpec((D, D), lambda i: (0, 0))
    bias = pl.BlockSpec((1, D), lambda i: (0, 0))
    return pl.pallas_call(
        _tc_body,
        grid=(N // R,),
        in_specs=[blk(D), blk(CW), blk(D), blk(CW), blk(D),
                  full, full, full, full, bias, bias],
        out_specs=blk(D),
        out_shape=jax.ShapeDtypeStruct((N, D), jnp.float32),
    )(acc_in, cnt_in, acc_out, cnt_out, x,
      wli_t, wlo_t, wri_t, wro_t, bli, blo)


def kernel(x, ei, Wl_in, bl_in, Wr_in, Wl_out, bl_out, Wr_out):
    acc, cnt = _sc_aggregate(x, ei)
    return _tc_combine(
        acc[0], cnt[0], acc[1], cnt[1], x,
        Wl_in.T, Wl_out.T, Wr_in.T, Wr_out.T,
        bl_in.reshape(1, D), bl_out.reshape(1, D))


# speedup vs baseline: 4.0336x; 4.0336x over previous
import functools

import jax
import jax.numpy as jnp
from jax import lax
from jax.experimental import pallas as pl
from jax.experimental.pallas import tpu as pltpu
from jax.experimental.pallas import tpu_sc as plsc

N = 10000
E = 320000
D = 128

NC = 2
NS = 16
L = 16

EPW = E // NS
K = 80
NCHUNK = EPW // K
RPT = 624
TAIL = N - NS * RPT


def _sc_body(x_hbm, e0_hbm, e1_hbm, zr_hbm, on_hbm, acc_hbm, cnt_hbm,
             acc_sh, gidx, sidx, rows, ones, sem):
    c = lax.axis_index("c")
    s = lax.axis_index("s")
    r0 = s * RPT

    def zero_acc():
        pltpu.sync_copy(zr_hbm, rows)
        for m in range(7):
            pltpu.sync_copy(rows, acc_sh.at[pl.ds(r0 + m * K, K)])
        pltpu.sync_copy(rows.at[pl.ds(0, RPT - 7 * K)],
                        acc_sh.at[pl.ds(r0 + 7 * K, RPT - 7 * K)])

        @pl.when(s == NS - 1)
        def _():
            pltpu.sync_copy(rows.at[pl.ds(0, TAIL)],
                            acc_sh.at[pl.ds(NS * RPT, TAIL)])

    def copy_out(dst_hbm):
        for m in range(7):
            pltpu.sync_copy(acc_sh.at[pl.ds(r0 + m * K, K)], rows)
            pltpu.sync_copy(rows, dst_hbm.at[c, pl.ds(r0 + m * K, K)])
        w = RPT - 7 * K
        pltpu.sync_copy(acc_sh.at[pl.ds(r0 + 7 * K, w)], rows.at[pl.ds(0, w)])
        pltpu.sync_copy(rows.at[pl.ds(0, w)],
                        dst_hbm.at[c, pl.ds(r0 + 7 * K, w)])

        @pl.when(s == NS - 1)
        def _():
            pltpu.sync_copy(acc_sh.at[pl.ds(NS * RPT, TAIL)],
                            rows.at[pl.ds(0, TAIL)])
            pltpu.sync_copy(rows.at[pl.ds(0, TAIL)],
                            dst_hbm.at[c, pl.ds(NS * RPT, TAIL)])

    def edge_loop(body):
        @pl.when(c == 0)
        def _():
            body(e0_hbm, e1_hbm)

        @pl.when(c == 1)
        def _():
            body(e1_hbm, e0_hbm)

    zero_acc()
    plsc.subcore_barrier()

    def acc_dir(g_hbm, s_hbm):
        def step(j, carry):
            base = s * EPW + j * K
            pltpu.sync_copy(g_hbm.at[pl.ds(base, K)], gidx)
            pltpu.sync_copy(s_hbm.at[pl.ds(base, K)], sidx)
            pltpu.async_copy(x_hbm.at[gidx], rows, sem).wait()
            pltpu.sync_copy(rows, acc_sh.at[sidx], add=True)
            return carry
        lax.fori_loop(0, NCHUNK, step, 0)

    edge_loop(acc_dir)
    plsc.subcore_barrier()
    copy_out(acc_hbm)
    plsc.subcore_barrier()

    zero_acc()
    pltpu.sync_copy(on_hbm, ones)
    plsc.subcore_barrier()

    def cnt_dir(g_hbm, s_hbm):
        def step(j, carry):
            base = s * EPW + j * K
            pltpu.sync_copy(s_hbm.at[pl.ds(base, K)], sidx)
            pltpu.sync_copy(ones, acc_sh.at[sidx], add=True)
            return carry
        lax.fori_loop(0, NCHUNK, step, 0)

    edge_loop(cnt_dir)
    plsc.subcore_barrier()
    copy_out(cnt_hbm)


@functools.cache
def _sc_aggregate():
    return pl.kernel(
        lambda *args: _sc_body(*args),
        out_type=(jax.ShapeDtypeStruct((NC, N, D), jnp.float32),
                  jax.ShapeDtypeStruct((NC, N, D), jnp.float32)),
        mesh=plsc.VectorSubcoreMesh(core_axis_name="c", subcore_axis_name="s",
                                    num_cores=NC, num_subcores=NS),
        scratch_types=[
            pltpu.VMEM_SHARED((N, D), jnp.float32),
            pltpu.VMEM((K,), jnp.int32),
            pltpu.VMEM((K,), jnp.int32),
            pltpu.VMEM((K, D), jnp.float32),
            pltpu.VMEM((K, D), jnp.float32),
            pltpu.SemaphoreType.DMA,
        ],
    )


R = 400


def _tc_body(acc_i, cnt_i, acc_o, cnt_o, x_ref,
             wli, wlo, wri, wro, bli, blo, out_ref):
    mi = acc_i[...] / jnp.maximum(cnt_i[:, 0:1], 1.0)
    mo = acc_o[...] / jnp.maximum(cnt_o[:, 0:1], 1.0)
    o = jnp.dot(mi, wli[...], preferred_element_type=jnp.float32)
    o = o + jnp.dot(mo, wlo[...], preferred_element_type=jnp.float32)
    o = o + jnp.dot(x_ref[...], wri[...] + wro[...],
                    preferred_element_type=jnp.float32)
    out_ref[...] = o + bli[0:1, :] + blo[0:1, :]


def _tc_combine(acc_in, cnt_in, acc_out, cnt_out, x,
                wli_t, wlo_t, wri_t, wro_t, bli, blo):
    blk = lambda w: pl.BlockSpec((R, w), lambda i: (i, 0))
    full = pl.BlockSpec((D, D), lambda i: (0, 0))
    bias = pl.BlockSpec((8, D), lambda i: (0, 0))
    return pl.pallas_call(
        _tc_body,
        grid=(N // R,),
        in_specs=[blk(D), blk(D), blk(D), blk(D), blk(D),
                  full, full, full, full, bias, bias],
        out_specs=blk(D),
        out_shape=jax.ShapeDtypeStruct((N, D), jnp.float32),
    )(acc_in, cnt_in, acc_out, cnt_out, x,
      wli_t, wlo_t, wri_t, wro_t, bli, blo)


def kernel(x, ei, Wl_in, bl_in, Wr_in, Wl_out, bl_out, Wr_out):
    zr = jnp.zeros((K, D), jnp.float32)
    on = jnp.ones((K, D), jnp.float32)
    acc, cnt = _sc_aggregate()(x, ei[0], ei[1], zr, on)
    return _tc_combine(
        acc[0], cnt[0], acc[1], cnt[1], x,
        Wl_in.T, Wl_out.T, Wr_in.T, Wr_out.T,
        jnp.broadcast_to(bl_in.reshape(1, D), (8, D)),
        jnp.broadcast_to(bl_out.reshape(1, D), (8, D)))
